# Initial kernel scaffold; baseline (speedup 1.0000x reference)
import jax
import jax.numpy as jnp
from jax.experimental import pallas as pl


def kernel(t1, t2, pos_w, pos_b, ln1_g, ln1_b, qkv_w, qkv_b, lepe_w, lepe_b, wo_w, wo_b, fc1_w, fc1_b, dw_w, dw_b, fc2_w, fc2_b, cc_w, cc_b, cn_g, cn_b):
    raise NotImplementedError("write your pallas kernel here")


# speedup vs baseline: 1.9621x; 1.9621x over previous
import jax
import jax.numpy as jnp
from jax.experimental import pallas as pl
from jax.experimental.pallas import tpu as pltpu

DIM = 384
QK = 384
HEADS = 8
NWIN = 8
TOPK = 2
SCALE = QK ** -0.5
HD = QK // HEADS
F32 = jnp.float32
BF16 = jnp.bfloat16


def _shift2(x, oy, ox):
    H, W, C = x.shape
    if oy > 0:
        x = jnp.concatenate([x[oy:], jnp.zeros((oy, W, C), x.dtype)], axis=0)
    elif oy < 0:
        x = jnp.concatenate([jnp.zeros((-oy, W, C), x.dtype), x[:H + oy]], axis=0)
    if ox > 0:
        x = jnp.concatenate([x[:, ox:], jnp.zeros((H, ox, C), x.dtype)], axis=1)
    elif ox < 0:
        x = jnp.concatenate([jnp.zeros((H, -ox, C), x.dtype), x[:, :W + ox]], axis=1)
    return x


def _dw_acc(x, w, r):
    k = 2 * r + 1
    out = None
    for ky in range(k):
        for kx in range(k):
            p = _shift2(x, ky - r, kx - r) * w[ky * k + kx][None, None, :]
            out = p if out is None else out + p
    return out


def _layernorm(xf, g, b, eps):
    mu = jnp.mean(xf, axis=-1, keepdims=True)
    xc = xf - mu
    var = jnp.mean(xc * xc, axis=-1, keepdims=True)
    return xc / jnp.sqrt(var + eps) * g + b


def _dw_acc_b16(x, w, r):
    return _dw_acc(x.astype(BF16).astype(F32), w.astype(BF16).astype(F32), r)


def _stage_a_kernel(x_ref, posw_ref, posb_ref, lng_ref, lnb_ref, qkvw_ref,
                    qkvb_ref, t_ref, nbar_ref, qkv_ref):
    x = x_ref[0]
    t = x + (_dw_acc_b16(x, posw_ref[...], 1) + posb_ref[0][None, None, :])
    t_ref[0] = t
    tf = t.reshape(1024, DIM)
    n = _layernorm(tf, lng_ref[0], lnb_ref[0], 1e-6)
    qkv = jnp.dot(n.astype(BF16), qkvw_ref[...], preferred_element_type=F32)
    qkv = qkv + qkvb_ref[0][None, :]
    qk2 = qkv[:, :2 * QK].reshape(8, 4, 32, 2 * QK)
    rs = qk2[:, 0] + qk2[:, 1] + qk2[:, 2] + qk2[:, 3]
    x4 = rs.reshape(8, 8, 4, 2 * QK)
    ws = x4[:, :, 0] + x4[:, :, 1] + x4[:, :, 2] + x4[:, :, 3]
    nbar_ref[0] = ws.reshape(64, 2 * QK) * (1.0 / 16.0)
    qkv_ref[0] = qkv.reshape(32, 32, 3 * DIM).astype(BF16)


def _route_kernel(nbar_ref, ridx_ref):
    wm = nbar_ref[...]
    col = jax.lax.broadcasted_iota(jnp.int32, (64, 64), 1)
    for d in range(8):
        q = (wm[d, :, :QK] * SCALE).astype(BF16)
        k = wm[(d + 4) % 8, :, QK:2 * QK].astype(BF16)
        logit = jax.lax.dot_general(q, k, (((1,), (1,)), ((), ())),
                                    preferred_element_type=F32)
        m1 = jnp.max(logit, axis=1, keepdims=True)
        i1 = jnp.min(jnp.where(logit == m1, col, 64), axis=1)
        l2 = jnp.where(col == i1[:, None], -1e30, logit)
        m2 = jnp.max(l2, axis=1, keepdims=True)
        i2 = jnp.min(jnp.where(l2 == m2, col, 64), axis=1)
        l3 = jnp.where(col == i2[:, None], -1e30, l2)
        m3 = jnp.max(l3, axis=1, keepdims=True)
        i3 = jnp.min(jnp.where(l3 == m3, col, 64), axis=1)
        lam = 0.5 * jnp.exp((m3 - m2)[:, 0] * 5000.0)
        ridx_ref[d, 0] = i1
        ridx_ref[d, 1] = i2
        ridx_ref[d, 2] = i3
        ridx_ref[d, 3] = (lam * 65536.0).astype(jnp.int32)


WB = 8


def _attn_kernel(ridx_ref, qr_ref, kT_ref, v_ref, o_ref):
    d = pl.program_id(0)
    wb = pl.program_id(1)
    rowh = jax.lax.broadcasted_iota(jnp.int32, (128, 256), 0) // 16
    colh = (jax.lax.broadcasted_iota(jnp.int32, (128, 256), 1) // 16) % 8
    valid = rowh == colh
    for j in range(WB):
        w = wb * WB + j
        i0 = ridx_ref[d, 0, w]
        i1 = ridx_ref[d, 1, w]
        i2 = ridx_ref[d, 2, w]
        lam = ridx_ref[d, 3, w].astype(F32) * (1.0 / 65536.0)
        q = qr_ref[0, j]
        k0 = kT_ref[0, :, pl.ds(i0 * 128, 128)]
        v0 = v_ref[0, pl.ds(i0 * 128, 128), :]

        def attn(ki, vi):
            kk = jnp.concatenate([k0, ki], axis=1)
            logits = jnp.dot(q, kk, preferred_element_type=F32) * SCALE
            logits = jnp.where(valid, logits, -1e30)
            m = jnp.max(logits, axis=1, keepdims=True)
            e = jnp.exp(logits - m)
            a = e / jnp.sum(e, axis=1, keepdims=True)
            vv = jnp.concatenate([v0, vi], axis=0)
            return jnp.dot(a.astype(BF16), vv, preferred_element_type=F32)

        o12 = attn(kT_ref[0, :, pl.ds(i1 * 128, 128)],
                   v_ref[0, pl.ds(i1 * 128, 128), :])
        o13 = attn(kT_ref[0, :, pl.ds(i2 * 128, 128)],
                   v_ref[0, pl.ds(i2 * 128, 128), :])
        o_ref[0, j] = (o12 + lam * (o13 - o12)).astype(BF16)


def _lepe_wo_kernel(t_ref, a_ref, v_ref, lw_ref, lb_ref, wow_ref, wob_ref, o_ref):
    vf = v_ref[0].astype(F32)
    lepe = _dw_acc_b16(vf, lw_ref[...], 2) + lb_ref[0][None, None, :]
    x = a_ref[0].astype(F32) + lepe
    xf = x.reshape(1024, DIM).astype(BF16)
    y = jnp.dot(xf, wow_ref[...], preferred_element_type=F32) + wob_ref[0][None, :]
    o_ref[0] = t_ref[0] + y.reshape(32, 32, DIM)


def _mlp_kernel(t_ref, lng_ref, lnb_ref, f1w_ref, f1b_ref, dww_ref, dwb_ref,
                f2w_ref, f2b_ref, o_ref):
    t = t_ref[0]
    n = _layernorm(t.reshape(1024, DIM), lng_ref[0], lnb_ref[0], 1e-6)
    h = jnp.dot(n.astype(BF16), f1w_ref[...], preferred_element_type=F32)
    h = (h + f1b_ref[0][None, :]).reshape(32, 32, 3 * DIM)
    h = _dw_acc_b16(h, dww_ref[...], 1) + dwb_ref[0][None, None, :]
    g = 0.5 * h * (1.0 + jax.lax.erf(h * (2.0 ** -0.5)))
    y = jnp.dot(g.reshape(1024, 3 * DIM).astype(BF16), f2w_ref[...],
                preferred_element_type=F32) + f2b_ref[0][None, :]
    o_ref[0] = t + y.reshape(32, 32, DIM)


def _final_kernel(t1_ref, t2_ref, ccw_ref, ccb_ref, cng_ref, cnb_ref, o_ref):
    x = jnp.concatenate([t1_ref[0], t2_ref[0]], axis=-1).astype(BF16)
    acc = jnp.zeros((1024, DIM), F32)
    for ky in range(3):
        for kx in range(3):
            oy, ox = ky - 1, kx - 1
            if oy == 0 and ox == 0:
                xs = x
            else:
                xs = _shift2(x, oy, ox)
            acc = acc + jnp.dot(xs.reshape(1024, 2 * DIM), ccw_ref[ky * 3 + kx],
                                preferred_element_type=F32)
    y = acc + ccb_ref[0][None, :]
    n = _layernorm(y, cng_ref[0], cnb_ref[0], 1e-5)
    o_ref[0] = jnp.maximum(n, 0.0).reshape(32, 32, DIM)


def _full(shape, dtype=F32):
    return pl.BlockSpec(shape, lambda *a: tuple(0 for _ in shape))


def kernel(t1, t2, pos_w, pos_b, ln1_g, ln1_b, qkv_w, qkv_b, lepe_w, lepe_b,
           wo_w, wo_b, fc1_w, fc1_b, dw_w, dw_b, fc2_w, fc2_b, cc_w, cc_b,
           cn_g, cn_b):
    x = jnp.concatenate([t1, t2], axis=0).transpose(0, 2, 3, 1)

    posw = pos_w.reshape(DIM, 9).T.reshape(9, DIM)
    lepew = lepe_w.reshape(DIM, 25).T.reshape(25, DIM)
    dww = dw_w.reshape(3 * DIM, 9).T.reshape(9, 3 * DIM)
    ccw = cc_w.transpose(2, 3, 1, 0).reshape(9, 2 * DIM, DIM).astype(BF16)
    r2 = lambda a: a.reshape(1, -1)

    t, nbar, qkv = pl.pallas_call(
        _stage_a_kernel,
        grid=(8,),
        in_specs=[
            pl.BlockSpec((1, 32, 32, DIM), lambda i: (i, 0, 0, 0)),
            _full((9, DIM)), _full((1, DIM)), _full((1, DIM)), _full((1, DIM)),
            _full((DIM, 3 * DIM)), _full((1, 3 * DIM)),
        ],
        out_specs=[
            pl.BlockSpec((1, 32, 32, DIM), lambda i: (i, 0, 0, 0)),
            pl.BlockSpec((1, 64, 2 * QK), lambda i: (i, 0, 0)),
            pl.BlockSpec((1, 32, 32, 3 * DIM), lambda i: (i, 0, 0, 0)),
        ],
        out_shape=[
            jax.ShapeDtypeStruct((8, 32, 32, DIM), F32),
            jax.ShapeDtypeStruct((8, 64, 2 * QK), F32),
            jax.ShapeDtypeStruct((8, 32, 32, 3 * DIM), BF16),
        ],
    )(x, posw, r2(pos_b), r2(ln1_g), r2(ln1_b), qkv_w.astype(BF16), r2(qkv_b))

    ridx = pl.pallas_call(
        _route_kernel,
        grid=(1,),
        in_specs=[_full((8, 64, 2 * QK))],
        out_specs=_full((8, 4, 64)),
        out_shape=jax.ShapeDtypeStruct((8, 4, 64), jnp.int32),
    )(nbar)

    qkvwin = qkv.reshape(8, 8, 4, 8, 4, 3 * DIM).transpose(0, 1, 3, 2, 4, 5)
    qkvwin = qkvwin.reshape(8, 64, 16, 3 * DIM)
    q5 = qkvwin[..., :QK].reshape(8, 64, 16, HEADS, HD)
    qr = q5.transpose(0, 1, 3, 2, 4).reshape(8, 64, 128, HD)
    k5 = qkvwin[..., QK:2 * QK].reshape(8, 64, 16, HEADS, HD)
    kT = k5.transpose(0, 4, 1, 3, 2).reshape(8, HD, 64 * 128)
    v5 = qkvwin[..., 2 * QK:].reshape(8, 64, 16, HEADS, HD)
    vr = v5.transpose(0, 1, 3, 2, 4).reshape(8, 64 * 128, HD)
    v_sp = qkv[..., 2 * QK:]

    ao = pl.pallas_call(
        _attn_kernel,
        grid_spec=pltpu.PrefetchScalarGridSpec(
            num_scalar_prefetch=1,
            grid=(8, 64 // WB),
            in_specs=[
                pl.BlockSpec((1, WB, 128, HD), lambda d, w, r: (d, w, 0, 0)),
                pl.BlockSpec((1, HD, 64 * 128), lambda d, w, r: ((d + 4) % 8, 0, 0)),
                pl.BlockSpec((1, 64 * 128, HD), lambda d, w, r: ((d + 4) % 8, 0, 0)),
            ],
            out_specs=pl.BlockSpec((1, WB, 128, HD), lambda d, w, r: (d, w, 0, 0)),
        ),
        out_shape=jax.ShapeDtypeStruct((8, 64, 128, HD), BF16),
    )(ridx, qr, kT, vr)

    a_sp = ao.reshape(8, 64, HEADS, 16, HD).transpose(0, 1, 3, 2, 4)
    a_sp = a_sp.reshape(8, 8, 8, 4, 4, DIM).transpose(0, 1, 3, 2, 4, 5)
    a_sp = a_sp.reshape(8, 32, 32, DIM)

    t2nd = pl.pallas_call(
        _lepe_wo_kernel,
        grid=(8,),
        in_specs=[
            pl.BlockSpec((1, 32, 32, DIM), lambda i: (i, 0, 0, 0)),
            pl.BlockSpec((1, 32, 32, DIM), lambda i: (i, 0, 0, 0)),
            pl.BlockSpec((1, 32, 32, DIM), lambda i: ((i + 4) % 8, 0, 0, 0)),
            _full((25, DIM)), _full((1, DIM)), _full((DIM, DIM)), _full((1, DIM)),
        ],
        out_specs=pl.BlockSpec((1, 32, 32, DIM), lambda i: (i, 0, 0, 0)),
        out_shape=jax.ShapeDtypeStruct((8, 32, 32, DIM), F32),
    )(t, a_sp, v_sp, lepew, r2(lepe_b), wo_w.astype(BF16), r2(wo_b))

    t3 = pl.pallas_call(
        _mlp_kernel,
        grid=(8,),
        in_specs=[
            pl.BlockSpec((1, 32, 32, DIM), lambda i: (i, 0, 0, 0)),
            _full((1, DIM)), _full((1, DIM)),
            _full((DIM, 3 * DIM)), _full((1, 3 * DIM)),
            _full((9, 3 * DIM)), _full((1, 3 * DIM)),
            _full((3 * DIM, DIM)), _full((1, DIM)),
        ],
        out_specs=pl.BlockSpec((1, 32, 32, DIM), lambda i: (i, 0, 0, 0)),
        out_shape=jax.ShapeDtypeStruct((8, 32, 32, DIM), F32),
    )(t2nd, r2(ln1_g), r2(ln1_b), fc1_w.astype(BF16), r2(fc1_b), dww, r2(dw_b),
      fc2_w.astype(BF16), r2(fc2_b))

    out = pl.pallas_call(
        _final_kernel,
        grid=(4,),
        in_specs=[
            pl.BlockSpec((1, 32, 32, DIM), lambda i: (i, 0, 0, 0)),
            pl.BlockSpec((1, 32, 32, DIM), lambda i: (i, 0, 0, 0)),
            _full((9, 2 * DIM, DIM)), _full((1, DIM)), _full((1, DIM)), _full((1, DIM)),
        ],
        out_specs=pl.BlockSpec((1, 32, 32, DIM), lambda i: (i, 0, 0, 0)),
        out_shape=jax.ShapeDtypeStruct((4, 32, 32, DIM), F32),
    )(t3[:4], t3[4:], ccw, r2(cc_b), r2(cn_g), r2(cn_b))

    return out.transpose(0, 3, 1, 2)
